# baseline (speedup 1.0000x reference)
import functools

import jax
import jax.numpy as jnp
from jax import lax
from jax.experimental import pallas as pl
from jax.experimental.pallas import tpu as pltpu
from jax.experimental.pallas import tpu_sc as plsc

E_DIM = 512
N_REL = 600
BATCH = 512
NNEG = 32
NC = 2
NS = 16
NW = NC * NS
LANES = 16
P_T = BATCH
N_T = NNEG * BATCH
CH = 8
S = 3

_CP = P_T // NW
_CN = N_T // NW
_NPC = _CP // CH
_NNC = _CN // CH
_TOTAL = _NPC + _NNC


def _box_body(rhb, rhw, rhs, rtb, rtw, rts, out):
    def half(base_ref, width_ref, scale_ref):
        w = width_ref[...]
        step2 = jnp.abs(w) + 1e-8
        norm_volume = jnp.exp(jnp.mean(jnp.log(step2), axis=1, keepdims=True))
        wn = w / norm_volume
        sc = scale_ref[...]
        s = jnp.where(sc > 0, sc, jnp.exp(sc) - 1.0) + 1.0
        d = wn * s
        b = base_ref[...]
        c1 = b + d
        c2 = b - d
        return jnp.maximum(c1, c2), jnp.minimum(c1, c2)

    hu, hl = half(rhb, rhw, rhs)
    tu, tl = half(rtb, rtw, rts)
    out[:, 0 * E_DIM:1 * E_DIM] = hu
    out[:, 1 * E_DIM:2 * E_DIM] = hl
    out[:, 2 * E_DIM:3 * E_DIM] = tu
    out[:, 3 * E_DIM:4 * E_DIM] = tl


def _box_tables(rhb, rhw, rhs, rtb, rtw, rts):
    rows = 120
    grid = N_REL // rows
    full = lambda i: (i, 0)
    return pl.pallas_call(
        _box_body,
        grid=(grid,),
        in_specs=[
            pl.BlockSpec((rows, E_DIM), full),
            pl.BlockSpec((rows, E_DIM), full),
            pl.BlockSpec((rows, 1), full),
            pl.BlockSpec((rows, E_DIM), full),
            pl.BlockSpec((rows, E_DIM), full),
            pl.BlockSpec((rows, 1), full),
        ],
        out_specs=pl.BlockSpec((rows, 4 * E_DIM), full),
        out_shape=jax.ShapeDtypeStruct((N_REL, 4 * E_DIM), jnp.float32),
    )(rhb, rhw, rhs, rtb, rtw, rts)


def _mesh():
    return plsc.VectorSubcoreMesh(
        core_axis_name="c", subcore_axis_name="s", num_cores=NC, num_subcores=NS
    )


SE = 3
SR = 4


@functools.lru_cache(maxsize=None)
def _sc_kernel():
    scratch = (
        [pltpu.VMEM((_CP,), jnp.int32)] * 3
        + [pltpu.VMEM((_CN,), jnp.int32)] * 3
        + [pltpu.VMEM((CH, 2, E_DIM), jnp.float32)] * (2 * SE)
        + [pltpu.VMEM((CH, 2, 2, E_DIM), jnp.float32)] * SR
        + [pltpu.SemaphoreType.DMA] * (2 * SE + 2 * SR)
    )

    @functools.partial(
        pl.kernel,
        mesh=_mesh(),
        out_type=(
            jax.ShapeDtypeStruct((1, P_T, 2, E_DIM), jnp.float32),
            jax.ShapeDtypeStruct((NNEG, BATCH, 2, E_DIM), jnp.float32),
            jax.ShapeDtypeStruct((1, P_T, 2, 2, E_DIM), jnp.float32),
            jax.ShapeDtypeStruct((NNEG, BATCH, 2, 2, E_DIM), jnp.float32),
        ),
        scratch_types=scratch,
    )
    def k(hp_hbm, tp_hbm, rp_hbm, hn_hbm, tn_hbm, rn_hbm, ent2_hbm, boxes_hbm,
          pe_hbm, ne_hbm, pr_hbm, nr_hbm, *sc):
        hidx_p, tidx_p, ridx_p, hidx_n, tidx_n, ridx_n = sc[0:6]
        o = 6
        hb = sc[o:o + SE]; o += SE
        tb = sc[o:o + SE]; o += SE
        rb = sc[o:o + SR]; o += SR
        egs = sc[o:o + SE]; o += SE
        ews = sc[o:o + SE]; o += SE
        rgs = sc[o:o + SR]; o += SR
        rws = sc[o:o + SR]; o += SR

        wid = lax.axis_index("s") * NC + lax.axis_index("c")
        pltpu.sync_copy(hp_hbm.at[pl.ds(wid * _CP, _CP)], hidx_p)
        pltpu.sync_copy(tp_hbm.at[pl.ds(wid * _CP, _CP)], tidx_p)
        pltpu.sync_copy(rp_hbm.at[pl.ds(wid * _CP, _CP)], ridx_p)
        pltpu.sync_copy(hn_hbm.at[pl.ds(wid * _CN, _CN)], hidx_n)
        pltpu.sync_copy(tn_hbm.at[pl.ds(wid * _CN, _CN)], tidx_n)
        pltpu.sync_copy(rn_hbm.at[pl.ds(wid * _CN, _CN)], ridx_n)

        def fire_eg(jj, se, pos=False):
            idx_h, idx_t = (hidx_p, tidx_p) if pos else (hidx_n, tidx_n)
            off = jj * CH
            pltpu.async_copy(ent2_hbm.at[idx_h.at[pl.ds(off, CH)]], hb[se], egs[se])
            pltpu.async_copy(ent2_hbm.at[idx_t.at[pl.ds(off, CH)]], tb[se], egs[se])

        def fire_rg(jj, sr, pos=False):
            idx_r = ridx_p if pos else ridx_n
            pltpu.async_copy(boxes_hbm.at[idx_r.at[pl.ds(jj * CH, CH)]], rb[sr], rgs[sr])

        def wait_eg(se):
            pltpu.make_async_copy(ne_hbm.at[0, pl.ds(0, CH)], hb[se], egs[se]).wait()
            pltpu.make_async_copy(ne_hbm.at[0, pl.ds(0, CH)], tb[se], egs[se]).wait()

        def wait_rg(sr):
            pltpu.make_async_copy(nr_hbm.at[0, pl.ds(0, CH)], rb[sr], rgs[sr]).wait()

        def compute(se):
            def row(i, _):
                def vec(kk, _):
                    sl = pl.ds(kk * LANES, LANES)
                    hb[se][i, 0, sl] = hb[se][i, 0, sl] + tb[se][i, 1, sl]
                    hb[se][i, 1, sl] = tb[se][i, 0, sl] + hb[se][i, 1, sl]
                    return 0

                lax.fori_loop(0, E_DIM // LANES, vec, 0, unroll=4)
                return 0

            lax.fori_loop(0, CH, row, 0)

        def fire_ew(jj, se, pos=False):
            if pos:
                pltpu.async_copy(hb[se], pe_hbm.at[0, pl.ds(wid * _CP + jj * CH, CH)], ews[se])
            else:
                pltpu.async_copy(hb[se], ne_hbm.at[wid, pl.ds(jj * CH, CH)], ews[se])

        def fire_rw(jj, sr, pos=False):
            if pos:
                pltpu.async_copy(rb[sr], pr_hbm.at[0, pl.ds(wid * _CP + jj * CH, CH)], rws[sr])
            else:
                pltpu.async_copy(rb[sr], nr_hbm.at[wid, pl.ds(jj * CH, CH)], rws[sr])

        def drain_ew(se):
            pltpu.make_async_copy(hb[se], ne_hbm.at[0, pl.ds(0, CH)], ews[se]).wait()

        def drain_rw(sr):
            pltpu.make_async_copy(rb[sr], nr_hbm.at[0, pl.ds(0, CH)], rws[sr]).wait()

        def iter_j(j, jj, pos=False, jj2=None):
            se, sr = j % SE, j % SR
            wait_eg(se)
            compute(se)
            fire_ew(jj, se, pos)
            wait_rg(sr)
            fire_rw(jj, sr, pos)
            if j >= 1:
                drain_ew((j - 1) % SE)
            if j >= 2:
                drain_rw((j - 2) % SR)
            if jj2 is not None:
                fire_eg(jj2, (j + 2) % SE)
                fire_rg(jj2, (j + 2) % SR)

        fire_eg(0, 0, pos=True)
        fire_rg(0, 0, pos=True)
        fire_eg(1, 1, pos=True)
        fire_rg(1, 1, pos=True)
        iter_j(0, 0, pos=True, jj2=0)
        iter_j(1, 1, pos=True, jj2=1)
        iter_j(2, 0, jj2=2)
        iter_j(3, 1, jj2=3)
        iter_j(4, 2, jj2=4)
        iter_j(5, 3, jj2=5)

        def group(g, _):
            for b in range(12):
                jb = 12 * g + b + 6
                jj = jb - _NPC
                se, sr = (6 + b) % SE, (6 + b) % SR
                wait_eg(se)
                compute(se)
                fire_ew(jj, se)
                wait_rg(sr)
                fire_rw(jj, sr)
                drain_ew((5 + b) % SE)
                drain_rw((4 + b) % SR)

                @pl.when(jj + 2 < _NNC)
                def _():
                    fire_eg(jj + 2, (8 + b) % SE)
                    fire_rg(jj + 2, (8 + b) % SR)

                _ = _
            return 0

        lax.fori_loop(0, (_TOTAL - 6) // 12, group, 0)
        drain_ew(65 % SE)
        drain_rw(64 % SR)
        drain_rw(65 % SR)

    return k


def kernel(positives, negatives, r_head_base_points, r_head_widths,
           r_head_size_scales, r_tail_base_points, r_tail_widths,
           r_tail_size_scales, entity_bases, entity_bumps):
    boxes = _box_tables(r_head_base_points, r_head_widths, r_head_size_scales,
                        r_tail_base_points, r_tail_widths, r_tail_size_scales)
    boxes = boxes.reshape(N_REL, 2, 2, E_DIM)
    ent2 = jnp.concatenate([entity_bases, entity_bumps], axis=1)
    ent2 = ent2.reshape(-1, 2, E_DIM)

    def ids(tuples, col):
        return tuples[:, col, :].reshape(-1).astype(jnp.int32)

    hp, rp, tp = ids(positives, 0), ids(positives, 1), ids(positives, 2)
    hn, rn, tn = ids(negatives, 0), ids(negatives, 1), ids(negatives, 2)

    p_ent, n_ent, p_rel, n_rel = _sc_kernel()(hp, tp, rp, hn, tn, rn, ent2, boxes)
    return (p_ent, p_rel, n_ent, n_rel)
